# baseline (speedup 1.0000x reference)
import functools

import jax
import jax.numpy as jnp
import numpy as np
from jax import lax
from jax.experimental import pallas as pl
from jax.experimental.pallas import tpu as pltpu
from jax.experimental.pallas import tpu_sc as plsc

_H = _W = 48
_A = 9
_K = 20
_N = _A * _H * _W
_NC, _NS, _L = 2, 16, 16
_NW = _NC * _NS
_CH = 656
_NV = _CH // _L
_NPAD = _NW * _CH
_POS = 0.7
_NEG = 0.3
_INV_SCALE = 1.0 / 16.0


def _anchor_tables() -> np.ndarray:
    sizes = np.array([[2., 2.], [4., 4.], [8., 8.], [2., 4.], [4., 8.],
                      [8., 16.], [4., 2.], [8., 4.], [16., 8.]], np.float32)
    ys, xs = np.meshgrid(np.arange(_H, dtype=np.float32),
                         np.arange(_W, dtype=np.float32), indexing="ij")
    w = np.broadcast_to(sizes[:, 0][:, None, None], (_A, _H, _W))
    h = np.broadcast_to(sizes[:, 1][:, None, None], (_A, _H, _W))
    x0 = xs[None] - w / 2.0
    y0 = ys[None] - h / 2.0
    flat = np.stack([x0, y0, w, h], axis=-1).reshape(-1, 4).astype(np.float32)
    inside = ((flat[:, 0] >= 0) & (flat[:, 1] >= 0)
              & (flat[:, 0] + flat[:, 2] < _H) & (flat[:, 1] + flat[:, 3] < _W))
    a = flat * 16.0
    anc = np.zeros((5, _NPAD), np.float32)
    anc[0, :_N] = a[:, 0]
    anc[1, :_N] = a[:, 1]
    anc[2, :_N] = a[:, 2]
    anc[3, :_N] = a[:, 3]
    anc[2, _N:] = 1.0
    anc[3, _N:] = 1.0
    anc[4, :_N] = inside.astype(np.float32)
    return anc


_ANC = _anchor_tables()
_INSIDE5 = _ANC[4, :_N].reshape(1, _A, _H, _W, 1)

_MESH = plsc.VectorSubcoreMesh(core_axis_name="c", subcore_axis_name="s",
                               num_cores=_NC, num_subcores=_NS)


@functools.partial(
    pl.kernel,
    out_type=(
        jax.ShapeDtypeStruct((_N,), jnp.float32),
        jax.ShapeDtypeStruct((_N * 4,), jnp.float32),
        jax.ShapeDtypeStruct((_NW * _L,), jnp.int32),
    ),
    mesh=_MESH,
    compiler_params=pltpu.CompilerParams(needs_layout_passes=False),
    scratch_types=(
        pltpu.VMEM((5 * _CH,), jnp.float32),
        pltpu.VMEM((128,), jnp.float32),
        pltpu.VMEM((96,), jnp.float32),
        pltpu.VMEM((_CH,), jnp.float32),
        pltpu.VMEM((_CH * 4,), jnp.float32),
        pltpu.VMEM((_L,), jnp.int32),
    ),
)
def _pass_a(anc_hbm, gt_hbm,
            lbl_hbm, tgt_hbm, cnt_hbm,
            anc_v, gt_v, gd_v, lbl_v, tgt_v, cnt_v):
    wid = lax.axis_index("s") * _NC + lax.axis_index("c")
    base = wid * _CH
    for r in range(5):
        pltpu.sync_copy(anc_hbm.at[pl.ds(r * _NPAD + base, _CH)],
                        anc_v.at[pl.ds(r * _CH, _CH)])
    pltpu.sync_copy(gt_hbm, gt_v)
    for j in range(2):
        gx = gt_v[pl.ds(0 * 32 + j * _L, _L)]
        gy = gt_v[pl.ds(1 * 32 + j * _L, _L)]
        gw = gt_v[pl.ds(2 * 32 + j * _L, _L)]
        gh = gt_v[pl.ds(3 * 32 + j * _L, _L)]
        gd_v[pl.ds(0 * 32 + j * _L, _L)] = gx + gw
        gd_v[pl.ds(1 * 32 + j * _L, _L)] = gy + gh
        gd_v[pl.ds(2 * 32 + j * _L, _L)] = gw * gh
    lane = jnp.arange(_L, dtype=jnp.int32)
    gx_l = [gt_v[pl.ds(0 * 32 + j * _L, _L)] for j in range(2)]
    gy_l = [gt_v[pl.ds(1 * 32 + j * _L, _L)] for j in range(2)]
    gxe_l = [gd_v[pl.ds(0 * 32 + j * _L, _L)] for j in range(2)]
    gye_l = [gd_v[pl.ds(1 * 32 + j * _L, _L)] for j in range(2)]
    ga_l = [gd_v[pl.ds(2 * 32 + j * _L, _L)] for j in range(2)]

    def body(i, pos_run):
        sl = pl.ds(i * _L, _L)
        ax0 = anc_v[pl.ds(0 * _CH + i * _L, _L)]
        ay0 = anc_v[pl.ds(1 * _CH + i * _L, _L)]
        aw = anc_v[pl.ds(2 * _CH + i * _L, _L)]
        ah = anc_v[pl.ds(3 * _CH + i * _L, _L)]
        ins = anc_v[pl.ds(4 * _CH + i * _L, _L)]
        axe = ax0 + aw
        aye = ay0 + ah
        area_a = aw * ah
        maxov = jnp.zeros((_L,), jnp.float32)
        arg = jnp.zeros((_L,), jnp.int32)
        for k in range(_K):
            j, e = divmod(k, _L)
            gxk = gx_l[j][e]
            gyk = gy_l[j][e]
            gxek = gxe_l[j][e]
            gyek = gye_l[j][e]
            gak = ga_l[j][e]
            iw = jnp.maximum(jnp.minimum(axe, gxek) - jnp.maximum(ax0, gxk), 0.0)
            ih = jnp.maximum(jnp.minimum(aye, gyek) - jnp.maximum(ay0, gyk), 0.0)
            inter = iw * ih
            ov = inter / (area_a + gak - inter)
            upd = ov > maxov
            arg = jnp.where(upd, k, arg)
            maxov = jnp.maximum(maxov, ov)
        insb = ins > 0.5
        pos = insb & (maxov > _POS)
        lbl = jnp.where(insb & (maxov >= _POS), 1.0, -1.0).astype(jnp.float32)
        lbl = jnp.where(insb & (maxov <= _NEG), 0.0, lbl)
        pos_run = pos_run + jnp.sum(pos.astype(jnp.int32))
        gsx = plsc.load_gather(gt_v.at[pl.ds(0, 32)], [arg])
        gsy = plsc.load_gather(gt_v.at[pl.ds(32, 32)], [arg])
        gsw = plsc.load_gather(gt_v.at[pl.ds(64, 32)], [arg])
        gsh = plsc.load_gather(gt_v.at[pl.ds(96, 32)], [arg])
        tx = (ax0 - gsx) * _INV_SCALE
        ty = (ay0 - gsy) * _INV_SCALE
        tw = (aw - gsw) * _INV_SCALE
        th = (ah - gsh) * _INV_SCALE
        lbl_v[sl] = lbl
        col = (lane + i * _L) * 4
        plsc.store_scatter(tgt_v, [col], tx)
        plsc.store_scatter(tgt_v, [col + 1], ty)
        plsc.store_scatter(tgt_v, [col + 2], tw)
        plsc.store_scatter(tgt_v, [col + 3], th)
        return pos_run

    pos_run = lax.fori_loop(0, _NV, body, jnp.int32(0))
    cnt_v[...] = jnp.where(lane == 1, pos_run, 0).astype(jnp.int32)
    last = _N - (_NW - 1) * _CH

    @pl.when(wid < _NW - 1)
    def _():
        pltpu.sync_copy(lbl_v, lbl_hbm.at[pl.ds(base, _CH)])
        pltpu.sync_copy(tgt_v, tgt_hbm.at[pl.ds(base * 4, _CH * 4)])

    @pl.when(wid == _NW - 1)
    def _():
        pltpu.sync_copy(lbl_v.at[pl.ds(0, last)], lbl_hbm.at[pl.ds(base, last)])
        pltpu.sync_copy(tgt_v.at[pl.ds(0, last * 4)],
                        tgt_hbm.at[pl.ds(base * 4, last * 4)])

    pltpu.sync_copy(cnt_v, cnt_hbm.at[pl.ds(wid * _L, _L)])


_R = _N // 128
_UPPER = np.triu(np.ones((128, 128), np.float32))
_LSTRICT = np.tril(np.ones((_R, _R), np.float32), k=-1)


def _combine_tc(lbl_ref, cnt_ref, u_ref, l_ref, out_ref):
    lbl = lbl_ref[...]
    isneg = (lbl == 0.0).astype(jnp.float32)
    incl = jnp.dot(isneg, u_ref[...], preferred_element_type=jnp.float32)
    rowsum = incl[:, 127:128]
    rowpre = jnp.dot(l_ref[...], rowsum, preferred_element_type=jnp.float32)
    grank = incl + rowpre
    negtot = jnp.sum(isneg)
    cnt = cnt_ref[...]
    posmask = lax.broadcasted_iota(jnp.int32, (4, 128), 1) % 16 == 1
    postot = jnp.sum(jnp.where(posmask, cnt, 0)).astype(jnp.float32)
    cut = jnp.maximum(3.0 * postot, 1.0)
    dis = (isneg > 0.0) & (grank <= negtot - cut) & (negtot > cut)
    out_ref[...] = jnp.where(dis, -1.0, lbl)


def _pass_b(lbl, cnt):
    return pl.pallas_call(
        _combine_tc,
        out_shape=jax.ShapeDtypeStruct((_R, 128), jnp.float32),
    )(lbl.reshape(_R, 128), cnt.reshape(4, 128),
      jnp.asarray(_UPPER), jnp.asarray(_LSTRICT))


def kernel(cls_scores, gt_boxes):
    del cls_scores
    g = gt_boxes[0]
    gt4 = jnp.zeros((4, 32), jnp.float32).at[:, :_K].set(g.T).reshape(-1)
    lbl, tgt, cnt = _pass_a(jnp.asarray(_ANC.reshape(-1)), gt4)
    lblf = _pass_b(lbl, cnt)
    label_op = lblf.reshape(1, _A, _H, _W, 1)
    target_op = tgt.reshape(1, _A, _H, _W, 4) * jnp.asarray(_INSIDE5)
    return (label_op, target_op)


# speedup vs baseline: 1.1796x; 1.1796x over previous
import functools

import jax
import jax.numpy as jnp
import numpy as np
from jax import lax
from jax.experimental import pallas as pl
from jax.experimental.pallas import tpu as pltpu
from jax.experimental.pallas import tpu_sc as plsc

_H = _W = 48
_A = 9
_K = 20
_N = _A * _H * _W
_NC, _NS, _L = 2, 16, 16
_NW = _NC * _NS
_CH = 656
_NV = _CH // _L
_NPAD = _NW * _CH
_POS = 0.7
_NEG = 0.3
_INV_SCALE = 1.0 / 16.0


def _anchor_tables() -> np.ndarray:
    sizes = np.array([[2., 2.], [4., 4.], [8., 8.], [2., 4.], [4., 8.],
                      [8., 16.], [4., 2.], [8., 4.], [16., 8.]], np.float32)
    ys, xs = np.meshgrid(np.arange(_H, dtype=np.float32),
                         np.arange(_W, dtype=np.float32), indexing="ij")
    w = np.broadcast_to(sizes[:, 0][:, None, None], (_A, _H, _W))
    h = np.broadcast_to(sizes[:, 1][:, None, None], (_A, _H, _W))
    x0 = xs[None] - w / 2.0
    y0 = ys[None] - h / 2.0
    flat = np.stack([x0, y0, w, h], axis=-1).reshape(-1, 4).astype(np.float32)
    inside = ((flat[:, 0] >= 0) & (flat[:, 1] >= 0)
              & (flat[:, 0] + flat[:, 2] < _H) & (flat[:, 1] + flat[:, 3] < _W))
    a = flat * 16.0
    anc = np.zeros((5, _NPAD), np.float32)
    anc[0, :_N] = a[:, 0]
    anc[1, :_N] = a[:, 1]
    anc[2, :_N] = a[:, 2]
    anc[3, :_N] = a[:, 3]
    anc[2, _N:] = 1.0
    anc[3, _N:] = 1.0
    anc[4, :_N] = inside.astype(np.float32)
    return anc


_ANC = _anchor_tables()
_INSIDE5 = _ANC[4, :_N].reshape(1, _A, _H, _W, 1)

_MESH = plsc.VectorSubcoreMesh(core_axis_name="c", subcore_axis_name="s",
                               num_cores=_NC, num_subcores=_NS)


@functools.partial(
    pl.kernel,
    out_type=(
        jax.ShapeDtypeStruct((_N,), jnp.float32),
        jax.ShapeDtypeStruct((_N * 4,), jnp.float32),
        jax.ShapeDtypeStruct((_NW * _L,), jnp.int32),
    ),
    mesh=_MESH,
    compiler_params=pltpu.CompilerParams(needs_layout_passes=False),
    scratch_types=(
        pltpu.VMEM((5 * _CH,), jnp.float32),
        pltpu.VMEM((128,), jnp.float32),
        pltpu.VMEM((96,), jnp.float32),
        pltpu.VMEM((_CH,), jnp.float32),
        pltpu.VMEM((_CH * 4,), jnp.float32),
        pltpu.VMEM((_L,), jnp.int32),
    ),
)
def _pass_a(anc_hbm, gt_hbm,
            lbl_hbm, tgt_hbm, cnt_hbm,
            anc_v, gt_v, gd_v, lbl_v, tgt_v, cnt_v):
    wid = lax.axis_index("s") * _NC + lax.axis_index("c")
    base = wid * _CH
    for r in range(5):
        pltpu.sync_copy(anc_hbm.at[pl.ds(r * _NPAD + base, _CH)],
                        anc_v.at[pl.ds(r * _CH, _CH)])
    pltpu.sync_copy(gt_hbm, gt_v)
    for j in range(2):
        gx = gt_v[pl.ds(0 * 32 + j * _L, _L)]
        gy = gt_v[pl.ds(1 * 32 + j * _L, _L)]
        gw = gt_v[pl.ds(2 * 32 + j * _L, _L)]
        gh = gt_v[pl.ds(3 * 32 + j * _L, _L)]
        gd_v[pl.ds(0 * 32 + j * _L, _L)] = gx + gw
        gd_v[pl.ds(1 * 32 + j * _L, _L)] = gy + gh
        gd_v[pl.ds(2 * 32 + j * _L, _L)] = gw * gh
    lane = jnp.arange(_L, dtype=jnp.int32)
    gx_l = [gt_v[pl.ds(0 * 32 + j * _L, _L)] for j in range(2)]
    gy_l = [gt_v[pl.ds(1 * 32 + j * _L, _L)] for j in range(2)]
    gxe_l = [gd_v[pl.ds(0 * 32 + j * _L, _L)] for j in range(2)]
    gye_l = [gd_v[pl.ds(1 * 32 + j * _L, _L)] for j in range(2)]
    ga_l = [gd_v[pl.ds(2 * 32 + j * _L, _L)] for j in range(2)]

    def body(i, pos_run):
        sl = pl.ds(i * _L, _L)
        ax0 = anc_v[pl.ds(0 * _CH + i * _L, _L)]
        ay0 = anc_v[pl.ds(1 * _CH + i * _L, _L)]
        aw = anc_v[pl.ds(2 * _CH + i * _L, _L)]
        ah = anc_v[pl.ds(3 * _CH + i * _L, _L)]
        ins = anc_v[pl.ds(4 * _CH + i * _L, _L)]
        axe = ax0 + aw
        aye = ay0 + ah
        area_a = aw * ah
        maxov = jnp.zeros((_L,), jnp.float32)
        arg = jnp.zeros((_L,), jnp.int32)
        for k in range(_K):
            j, e = divmod(k, _L)
            gxk = gx_l[j][e]
            gyk = gy_l[j][e]
            gxek = gxe_l[j][e]
            gyek = gye_l[j][e]
            gak = ga_l[j][e]
            iw = jnp.maximum(jnp.minimum(axe, gxek) - jnp.maximum(ax0, gxk), 0.0)
            ih = jnp.maximum(jnp.minimum(aye, gyek) - jnp.maximum(ay0, gyk), 0.0)
            inter = iw * ih
            ov = inter / (area_a + gak - inter)
            upd = ov > maxov
            arg = jnp.where(upd, k, arg)
            maxov = jnp.maximum(maxov, ov)
        insb = ins > 0.5
        pos = insb & (maxov > _POS)
        lbl = jnp.where(insb & (maxov >= _POS), 1.0, -1.0).astype(jnp.float32)
        lbl = jnp.where(insb & (maxov <= _NEG), 0.0, lbl)
        pos_run = pos_run + jnp.sum(pos.astype(jnp.int32))
        gsx = plsc.load_gather(gt_v.at[pl.ds(0, 32)], [arg])
        gsy = plsc.load_gather(gt_v.at[pl.ds(32, 32)], [arg])
        gsw = plsc.load_gather(gt_v.at[pl.ds(64, 32)], [arg])
        gsh = plsc.load_gather(gt_v.at[pl.ds(96, 32)], [arg])
        tx = jnp.where(insb, (ax0 - gsx) * _INV_SCALE, 0.0)
        ty = jnp.where(insb, (ay0 - gsy) * _INV_SCALE, 0.0)
        tw = jnp.where(insb, (aw - gsw) * _INV_SCALE, 0.0)
        th = jnp.where(insb, (ah - gsh) * _INV_SCALE, 0.0)
        lbl_v[sl] = lbl
        col = (lane + i * _L) * 4
        plsc.store_scatter(tgt_v, [col], tx)
        plsc.store_scatter(tgt_v, [col + 1], ty)
        plsc.store_scatter(tgt_v, [col + 2], tw)
        plsc.store_scatter(tgt_v, [col + 3], th)
        return pos_run

    pos_run = lax.fori_loop(0, _NV, body, jnp.int32(0))
    cnt_v[...] = jnp.where(lane == 1, pos_run, 0).astype(jnp.int32)
    last = _N - (_NW - 1) * _CH

    @pl.when(wid < _NW - 1)
    def _():
        pltpu.sync_copy(lbl_v, lbl_hbm.at[pl.ds(base, _CH)])
        pltpu.sync_copy(tgt_v, tgt_hbm.at[pl.ds(base * 4, _CH * 4)])

    @pl.when(wid == _NW - 1)
    def _():
        pltpu.sync_copy(lbl_v.at[pl.ds(0, last)], lbl_hbm.at[pl.ds(base, last)])
        pltpu.sync_copy(tgt_v.at[pl.ds(0, last * 4)],
                        tgt_hbm.at[pl.ds(base * 4, last * 4)])

    pltpu.sync_copy(cnt_v, cnt_hbm.at[pl.ds(wid * _L, _L)])


_R = _N // 128
_UPPER = np.triu(np.ones((128, 128), np.float32))
_LSTRICT = np.tril(np.ones((_R, _R), np.float32), k=-1)


def _combine_tc(lbl_ref, cnt_ref, u_ref, l_ref, out_ref):
    lbl = lbl_ref[...]
    isneg = (lbl == 0.0).astype(jnp.float32)
    incl = jnp.dot(isneg, u_ref[...], preferred_element_type=jnp.float32)
    rowsum = incl[:, 127:128]
    rowpre = jnp.dot(l_ref[...], rowsum, preferred_element_type=jnp.float32)
    grank = incl + rowpre
    negtot = jnp.sum(isneg)
    cnt = cnt_ref[...]
    posmask = lax.broadcasted_iota(jnp.int32, (4, 128), 1) % 16 == 1
    postot = jnp.sum(jnp.where(posmask, cnt, 0)).astype(jnp.float32)
    cut = jnp.maximum(3.0 * postot, 1.0)
    dis = (isneg > 0.0) & (grank <= negtot - cut) & (negtot > cut)
    out_ref[...] = jnp.where(dis, -1.0, lbl)


def _pass_b(lbl, cnt):
    return pl.pallas_call(
        _combine_tc,
        out_shape=jax.ShapeDtypeStruct((_R, 128), jnp.float32),
    )(lbl.reshape(_R, 128), cnt.reshape(4, 128),
      jnp.asarray(_UPPER), jnp.asarray(_LSTRICT))


def kernel(cls_scores, gt_boxes):
    del cls_scores
    g = gt_boxes[0]
    gt4 = jnp.zeros((4, 32), jnp.float32).at[:, :_K].set(g.T).reshape(-1)
    lbl, tgt, cnt = _pass_a(jnp.asarray(_ANC.reshape(-1)), gt4)
    lblf = _pass_b(lbl, cnt)
    label_op = lblf.reshape(1, _A, _H, _W, 1)
    target_op = tgt.reshape(1, _A, _H, _W, 4)
    return (label_op, target_op)


# speedup vs baseline: 1.5679x; 1.3291x over previous
import functools

import jax
import jax.numpy as jnp
import numpy as np
from jax import lax
from jax.experimental import pallas as pl
from jax.experimental.pallas import tpu as pltpu
from jax.experimental.pallas import tpu_sc as plsc

_H = _W = 48
_A = 9
_K = 20
_N = _A * _H * _W
_NC, _NS, _L = 2, 16, 16
_NW = _NC * _NS
_CH = 656
_NV = _CH // _L
_NPAD = _NW * _CH
_POS = 0.7
_NEG = 0.3
_INV_SCALE = 1.0 / 16.0


def _anchor_tables() -> np.ndarray:
    sizes = np.array([[2., 2.], [4., 4.], [8., 8.], [2., 4.], [4., 8.],
                      [8., 16.], [4., 2.], [8., 4.], [16., 8.]], np.float32)
    ys, xs = np.meshgrid(np.arange(_H, dtype=np.float32),
                         np.arange(_W, dtype=np.float32), indexing="ij")
    w = np.broadcast_to(sizes[:, 0][:, None, None], (_A, _H, _W))
    h = np.broadcast_to(sizes[:, 1][:, None, None], (_A, _H, _W))
    x0 = xs[None] - w / 2.0
    y0 = ys[None] - h / 2.0
    flat = np.stack([x0, y0, w, h], axis=-1).reshape(-1, 4).astype(np.float32)
    inside = ((flat[:, 0] >= 0) & (flat[:, 1] >= 0)
              & (flat[:, 0] + flat[:, 2] < _H) & (flat[:, 1] + flat[:, 3] < _W))
    a = flat * 16.0
    anc = np.zeros((5, _NPAD), np.float32)
    anc[0, :_N] = a[:, 0]
    anc[1, :_N] = a[:, 1]
    anc[2, :_N] = a[:, 2]
    anc[3, :_N] = a[:, 3]
    anc[2, _N:] = 1.0
    anc[3, _N:] = 1.0
    anc[4, :_N] = inside.astype(np.float32)
    return anc


_ANC = _anchor_tables()
_INSIDE5 = _ANC[4, :_N].reshape(1, _A, _H, _W, 1)

_MESH = plsc.VectorSubcoreMesh(core_axis_name="c", subcore_axis_name="s",
                               num_cores=_NC, num_subcores=_NS)


@functools.partial(
    pl.kernel,
    out_type=(
        jax.ShapeDtypeStruct((_N,), jnp.float32),
        jax.ShapeDtypeStruct((_N,), jnp.float32),
        jax.ShapeDtypeStruct((_N,), jnp.float32),
        jax.ShapeDtypeStruct((_N,), jnp.float32),
        jax.ShapeDtypeStruct((_N,), jnp.float32),
        jax.ShapeDtypeStruct((_NW * _L,), jnp.int32),
    ),
    mesh=_MESH,
    compiler_params=pltpu.CompilerParams(needs_layout_passes=False),
    scratch_types=(
        pltpu.VMEM((5 * _CH,), jnp.float32),
        pltpu.VMEM((128,), jnp.float32),
        pltpu.VMEM((128,), jnp.float32),
        pltpu.VMEM((96,), jnp.float32),
        pltpu.VMEM((_CH,), jnp.float32),
        pltpu.VMEM((_CH * 4,), jnp.float32),
        pltpu.VMEM((_L,), jnp.int32),
    ),
)
def _pass_a(anc_hbm, gt_hbm,
            lbl_hbm, tx_hbm, ty_hbm, tw_hbm, th_hbm, cnt_hbm,
            anc_v, graw_v, gt_v, gd_v, lbl_v, tgt_v, cnt_v):
    wid = lax.axis_index("s") * _NC + lax.axis_index("c")
    base = wid * _CH
    for r in range(5):
        pltpu.sync_copy(anc_hbm.at[pl.ds(r * _NPAD + base, _CH)],
                        anc_v.at[pl.ds(r * _CH, _CH)])
    pltpu.sync_copy(gt_hbm, graw_v)
    lane0 = jnp.arange(_L, dtype=jnp.int32)
    for c in range(4):
        for j in range(2):
            idx = (lane0 + j * _L) * 4 + c
            gt_v[pl.ds(c * 32 + j * _L, _L)] = plsc.load_gather(graw_v, [idx])
    for j in range(2):
        gx = gt_v[pl.ds(0 * 32 + j * _L, _L)]
        gy = gt_v[pl.ds(1 * 32 + j * _L, _L)]
        gw = gt_v[pl.ds(2 * 32 + j * _L, _L)]
        gh = gt_v[pl.ds(3 * 32 + j * _L, _L)]
        gd_v[pl.ds(0 * 32 + j * _L, _L)] = gx + gw
        gd_v[pl.ds(1 * 32 + j * _L, _L)] = gy + gh
        gd_v[pl.ds(2 * 32 + j * _L, _L)] = gw * gh
    lane = jnp.arange(_L, dtype=jnp.int32)
    gx_l = [gt_v[pl.ds(0 * 32 + j * _L, _L)] for j in range(2)]
    gy_l = [gt_v[pl.ds(1 * 32 + j * _L, _L)] for j in range(2)]
    gxe_l = [gd_v[pl.ds(0 * 32 + j * _L, _L)] for j in range(2)]
    gye_l = [gd_v[pl.ds(1 * 32 + j * _L, _L)] for j in range(2)]
    ga_l = [gd_v[pl.ds(2 * 32 + j * _L, _L)] for j in range(2)]

    def body(i, pos_run):
        sl = pl.ds(i * _L, _L)
        ax0 = anc_v[pl.ds(0 * _CH + i * _L, _L)]
        ay0 = anc_v[pl.ds(1 * _CH + i * _L, _L)]
        aw = anc_v[pl.ds(2 * _CH + i * _L, _L)]
        ah = anc_v[pl.ds(3 * _CH + i * _L, _L)]
        ins = anc_v[pl.ds(4 * _CH + i * _L, _L)]
        axe = ax0 + aw
        aye = ay0 + ah
        area_a = aw * ah
        maxov = jnp.zeros((_L,), jnp.float32)
        arg = jnp.zeros((_L,), jnp.int32)
        for k in range(_K):
            j, e = divmod(k, _L)
            gxk = gx_l[j][e]
            gyk = gy_l[j][e]
            gxek = gxe_l[j][e]
            gyek = gye_l[j][e]
            gak = ga_l[j][e]
            iw = jnp.maximum(jnp.minimum(axe, gxek) - jnp.maximum(ax0, gxk), 0.0)
            ih = jnp.maximum(jnp.minimum(aye, gyek) - jnp.maximum(ay0, gyk), 0.0)
            inter = iw * ih
            ov = inter / (area_a + gak - inter)
            upd = ov > maxov
            arg = jnp.where(upd, k, arg)
            maxov = jnp.maximum(maxov, ov)
        insb = ins > 0.5
        pos = insb & (maxov > _POS)
        lbl = jnp.where(insb & (maxov >= _POS), 1.0, -1.0).astype(jnp.float32)
        lbl = jnp.where(insb & (maxov <= _NEG), 0.0, lbl)
        pos_run = pos_run + jnp.sum(pos.astype(jnp.int32))
        gsx = plsc.load_gather(gt_v.at[pl.ds(0, 32)], [arg])
        gsy = plsc.load_gather(gt_v.at[pl.ds(32, 32)], [arg])
        gsw = plsc.load_gather(gt_v.at[pl.ds(64, 32)], [arg])
        gsh = plsc.load_gather(gt_v.at[pl.ds(96, 32)], [arg])
        tx = jnp.where(insb, (ax0 - gsx) * _INV_SCALE, 0.0)
        ty = jnp.where(insb, (ay0 - gsy) * _INV_SCALE, 0.0)
        tw = jnp.where(insb, (aw - gsw) * _INV_SCALE, 0.0)
        th = jnp.where(insb, (ah - gsh) * _INV_SCALE, 0.0)
        lbl_v[sl] = lbl
        tgt_v[pl.ds(0 * _CH + i * _L, _L)] = tx
        tgt_v[pl.ds(1 * _CH + i * _L, _L)] = ty
        tgt_v[pl.ds(2 * _CH + i * _L, _L)] = tw
        tgt_v[pl.ds(3 * _CH + i * _L, _L)] = th
        return pos_run

    pos_run = lax.fori_loop(0, _NV, body, jnp.int32(0))
    cnt_v[...] = jnp.where(lane == 1, pos_run, 0).astype(jnp.int32)
    last = _N - (_NW - 1) * _CH

    t_hbms = (tx_hbm, ty_hbm, tw_hbm, th_hbm)

    @pl.when(wid < _NW - 1)
    def _():
        pltpu.sync_copy(lbl_v, lbl_hbm.at[pl.ds(base, _CH)])
        for c in range(4):
            pltpu.sync_copy(tgt_v.at[pl.ds(c * _CH, _CH)],
                            t_hbms[c].at[pl.ds(base, _CH)])

    @pl.when(wid == _NW - 1)
    def _():
        pltpu.sync_copy(lbl_v.at[pl.ds(0, last)], lbl_hbm.at[pl.ds(base, last)])
        for c in range(4):
            pltpu.sync_copy(tgt_v.at[pl.ds(c * _CH, last)],
                            t_hbms[c].at[pl.ds(base, last)])

    pltpu.sync_copy(cnt_v, cnt_hbm.at[pl.ds(wid * _L, _L)])


_R = _N // 128
_UPPER = np.triu(np.ones((128, 128), np.float32))
_LSTRICT = np.tril(np.ones((_R, _R), np.float32), k=-1)


def _combine_tc(lbl_ref, cnt_ref, u_ref, l_ref, out_ref):
    lbl = lbl_ref[...]
    isneg = (lbl == 0.0).astype(jnp.float32)
    incl = jnp.dot(isneg, u_ref[...], preferred_element_type=jnp.float32)
    rowsum = incl[:, 127:128]
    rowpre = jnp.dot(l_ref[...], rowsum, preferred_element_type=jnp.float32)
    grank = incl + rowpre
    negtot = jnp.sum(isneg)
    cnt = cnt_ref[...]
    posmask = lax.broadcasted_iota(jnp.int32, (4, 128), 1) % 16 == 1
    postot = jnp.sum(jnp.where(posmask, cnt, 0)).astype(jnp.float32)
    cut = jnp.maximum(3.0 * postot, 1.0)
    dis = (isneg > 0.0) & (grank <= negtot - cut) & (negtot > cut)
    out_ref[...] = jnp.where(dis, -1.0, lbl)


def _pass_b(lbl, cnt):
    return pl.pallas_call(
        _combine_tc,
        out_shape=jax.ShapeDtypeStruct((_R, 128), jnp.float32),
    )(lbl.reshape(_R, 128), cnt.reshape(4, 128),
      jnp.asarray(_UPPER), jnp.asarray(_LSTRICT))


def kernel(cls_scores, gt_boxes):
    del cls_scores
    gt_flat = jnp.concatenate([gt_boxes.reshape(-1),
                               jnp.zeros((128 - 4 * _K,), jnp.float32)])
    lbl, tx, ty, tw, th, cnt = _pass_a(jnp.asarray(_ANC.reshape(-1)), gt_flat)
    lblf = _pass_b(lbl, cnt)
    label_op = lblf.reshape(1, _A, _H, _W, 1)
    target_op = jnp.stack([tx, ty, tw, th], axis=-1).reshape(1, _A, _H, _W, 4)
    return (label_op, target_op)
